# baseline (speedup 1.0000x reference)
import functools

import jax
import jax.numpy as jnp
from jax import lax
from jax.experimental import pallas as pl
from jax.experimental.pallas import tpu as pltpu
from jax.experimental.pallas import tpu_sc as plsc

_V = 1000000
_D = 32
_B = 16384
_L = 16
_NC = 2
_NS = 16
_NW = _NC * _NS
_IC = 128
_NCH = _B // _IC
_FIRE = 16


def _sc_gather(movies_hbm, tab_hbm, part_hbm, idx_v, gbuf_v, part_v, sem):
    cid = lax.axis_index("c")
    sid = lax.axis_index("s")
    wid = cid * _NS + sid

    pltpu.sync_copy(movies_hbm, idx_v)

    accs = [jnp.zeros((_L,), jnp.float32) for _ in range(4)]

    def _reduce_row(j, accs):
        def red(i, a):
            a0, a1, a2, a3 = a
            a0 = a0 + gbuf_v[j, pl.ds(i * 4 * _L, _L)]
            a1 = a1 + gbuf_v[j, pl.ds((i * 4 + 1) * _L, _L)]
            a2 = a2 + gbuf_v[j, pl.ds((i * 4 + 2) * _L, _L)]
            a3 = a3 + gbuf_v[j, pl.ds((i * 4 + 3) * _L, _L)]
            return (a0, a1, a2, a3)
        return list(lax.fori_loop(0, _IC // (4 * _L), red, tuple(accs)))

    del _reduce_row

    part_v[...] = (accs[0] + accs[1]) + (accs[2] + accs[3])
    pltpu.sync_copy(part_v, part_hbm.at[wid])


def _tc_final(p_ref, userT_ref, out_ref):
    s = jnp.sum(p_ref[...], axis=1, keepdims=True)
    out_ref[...] = s * userT_ref[...]


@jax.jit
def _run(movies, movie_table, user_table):
    tabT = movie_table.T
    mesh = plsc.VectorSubcoreMesh(core_axis_name="c", subcore_axis_name="s")
    part = pl.kernel(
        _sc_gather,
        out_type=jax.ShapeDtypeStruct((_NW, _L), jnp.float32),
        mesh=mesh,
        compiler_params=pltpu.CompilerParams(use_tc_tiling_on_sc=False),
        scratch_types=[
            pltpu.VMEM((_NCH, _IC), jnp.int32),
            pltpu.VMEM((_NCH, _IC), jnp.float32),
            pltpu.VMEM((_L,), jnp.float32),
            pltpu.SemaphoreType.DMA,
        ],
    )(movies.reshape(_NCH, _IC), movie_table)

    userT = user_table.T
    out = pl.pallas_call(
        _tc_final,
        in_specs=[
            pl.BlockSpec((_NW, _L), lambda: (0, 0)),
            pl.BlockSpec((_D, 1), lambda: (0, 0)),
        ],
        out_specs=pl.BlockSpec((_D, 1), lambda: (0, 0)),
        out_shape=jax.ShapeDtypeStruct((_D, 1), jnp.float32),
    )(part, userT)
    return out.reshape(1, _D)


def kernel(users, movies, movie_table, user_table):
    return _run(movies.astype(jnp.int32), movie_table, user_table)


# speedup vs baseline: 3.5018x; 3.5018x over previous
import functools

import jax
import jax.numpy as jnp
from jax import lax
from jax.experimental import pallas as pl
from jax.experimental.pallas import tpu as pltpu
from jax.experimental.pallas import tpu_sc as plsc

_V = 1000000
_D = 32
_B = 16384
_L = 16
_NC = 2
_NS = 16
_BPW = _B // (_NC * _NS)
_IC = 128
_NIC = _BPW // _IC

_C = 8192
_NSTEP = -(-_V // _C)
_HV = _NSTEP * _C
_ZW = 10
_ZCH = _V // _ZW
_ZCHUNK = 20000
_ZITER = _ZCH // _ZCHUNK


def _sc_hist(movies_hbm, zeros_hbm, hist_hbm, idx_v, ones_v, buf_v, hist_sh):
    cid = lax.axis_index("c")
    sid = lax.axis_index("s")

    @pl.when(sid < _ZW)
    def _():
        pltpu.sync_copy(zeros_hbm, buf_v)
        for z in range(_ZITER):
            pltpu.sync_copy(
                buf_v, hist_sh.at[pl.ds(sid * _ZCH + z * _ZCHUNK, _ZCHUNK)])

    base = (cid * _NS + sid) * _BPW
    for j in range(_NIC):
        pltpu.sync_copy(movies_hbm.at[pl.ds(base + j * _IC, _IC)],
                        idx_v.at[j])

    for k in range(_IC // _L):
        ones_v[pl.ds(k * _L, _L)] = jnp.full((_L,), 1.0, jnp.float32)

    plsc.subcore_barrier()

    for j in range(_NIC):
        pltpu.sync_copy(ones_v, hist_sh.at[idx_v.at[j]], add=True)

    plsc.subcore_barrier()

    @pl.when(sid < _ZW)
    def _():
        for z in range(_ZITER):
            off = sid * _ZCH + z * _ZCHUNK
            pltpu.sync_copy(hist_sh.at[pl.ds(off, _ZCHUNK)], buf_v)
            pltpu.sync_copy(buf_v, hist_hbm.at[pl.ds(cid * _HV + off, _ZCHUNK)])


def _tc_matvec(tabT_ref, h0_ref, h1_ref, userT_ref, out_ref, acc_ref):
    i = pl.program_id(0)

    @pl.when(i == 0)
    def _():
        acc_ref[...] = jnp.zeros_like(acc_ref)

    h = h0_ref[...] + h1_ref[...]
    prod = tabT_ref[...] * h[None, :]
    cols = i * _C + lax.broadcasted_iota(jnp.int32, (_D, _C), 1)
    prod = jnp.where(cols < _V, prod, 0.0)
    acc_ref[...] += jnp.sum(prod.reshape(_D, _C // 128, 128), axis=1)

    @pl.when(i == _NSTEP - 1)
    def _():
        s = jnp.sum(acc_ref[...], axis=1, keepdims=True)
        out_ref[...] = s * userT_ref[...]


@jax.jit
def _run(movies, movie_table, user_table):
    zeros = jnp.zeros((_ZCHUNK,), jnp.float32)
    mesh = plsc.VectorSubcoreMesh(core_axis_name="c", subcore_axis_name="s")
    hist = pl.kernel(
        _sc_hist,
        out_type=jax.ShapeDtypeStruct((_NC * _HV,), jnp.float32),
        mesh=mesh,
        scratch_types=[
            pltpu.VMEM((_NIC, _IC), jnp.int32),
            pltpu.VMEM((_IC,), jnp.float32),
            pltpu.VMEM((_ZCHUNK,), jnp.float32),
            pltpu.VMEM_SHARED((_V,), jnp.float32),
        ],
    )(movies, zeros)

    tabT = movie_table.T
    userT = user_table.T
    out = pl.pallas_call(
        _tc_matvec,
        grid=(_NSTEP,),
        in_specs=[
            pl.BlockSpec((_D, _C), lambda i: (0, i)),
            pl.BlockSpec((_C,), lambda i: (i,)),
            pl.BlockSpec((_C,), lambda i: (i + _NSTEP,)),
            pl.BlockSpec((_D, 1), lambda i: (0, 0)),
        ],
        out_specs=pl.BlockSpec((_D, 1), lambda i: (0, 0)),
        out_shape=jax.ShapeDtypeStruct((_D, 1), jnp.float32),
        scratch_shapes=[pltpu.VMEM((_D, 128), jnp.float32)],
    )(tabT, hist, hist, userT)
    return out.reshape(1, _D)


def kernel(users, movies, movie_table, user_table):
    return _run(movies.astype(jnp.int32), movie_table, user_table)
